# baseline (speedup 1.0000x reference)
import functools

import jax
import jax.numpy as jnp
from jax import lax
from jax.experimental import pallas as pl
from jax.experimental.pallas import tpu as pltpu
from jax.experimental.pallas import tpu_sc as plsc

R = 2048
RR = R * R
LEVELS = 8
N_LOC = 3
OUT_D = 64

NC = 2
NS = 16
NW = NC * NS

CHUNK = 128
G = CHUNK // 16
M = G * 12 * 16

_LOC_S1 = [17, 33, 65]
_LOC_BASE = [0, 3 * 17 * 17, 3 * 17 * 17 + 3 * 33 * 33]
LOC_PAIRS = _LOC_BASE[2] + 3 * 65 * 65
_STAGE = [(6, 3), (6, 6), (3, 22)]


def _encode_sc(px, py, pz, rows, lastv):
    n = px.shape[0]
    per_w = n // NW
    n_pairs = per_w // (2 * CHUNK)
    mesh = plsc.VectorSubcoreMesh(core_axis_name="c", subcore_axis_name="s")

    buf_types = [
        pltpu.VMEM((M,), jnp.int32),
        pltpu.VMEM((M,), jnp.int32),
        pltpu.VMEM((G * 4 * 16,), jnp.float32),
        pltpu.VMEM((M, 8), jnp.float32),
        pltpu.VMEM((CHUNK * OUT_D,), jnp.float32),
    ]

    @functools.partial(
        pl.kernel,
        mesh=mesh,
        out_type=jax.ShapeDtypeStruct((n * OUT_D,), jnp.float32),
        compiler_params=pltpu.CompilerParams(
            needs_layout_passes=False, use_tc_tiling_on_sc=False),
        scratch_types=[
            pltpu.VMEM((per_w,), jnp.float32),
            pltpu.VMEM((per_w,), jnp.float32),
            pltpu.VMEM((per_w,), jnp.float32),
            pltpu.VMEM((16,), jnp.float32),
            pltpu.VMEM((2 * LOC_PAIRS,), jnp.float32),
        ] + buf_types + buf_types + [
            pltpu.SemaphoreType.DMA,
            pltpu.SemaphoreType.DMA,
            pltpu.SemaphoreType.DMA,
            pltpu.SemaphoreType.DMA,
        ],
    )
    def enc(px_h, py_h, pz_h, rows_h, lastv_h, out_h,
            pxw, pyw, pzw, lastv_v, loc_v,
            idx0, off0, mk0, gath0, outv0,
            idx1, off1, mk1, gath1, outv1,
            gsem0, gsem1, osem0, osem1):
        wid = lax.axis_index("s") * NC + lax.axis_index("c")
        base = wid * per_w
        iota = lax.broadcasted_iota(jnp.int32, (16,), 0)
        pltpu.sync_copy(lastv_h, lastv_v)
        pltpu.sync_copy(px_h.at[pl.ds(base, per_w)], pxw)
        pltpu.sync_copy(py_h.at[pl.ds(base, per_w)], pyw)
        pltpu.sync_copy(pz_h.at[pl.ds(base, per_w)], pzw)
        last16 = lastv_v[...]

        bufs = ((idx0, off0, mk0, gath0, outv0, gsem0, osem0),
                (idx1, off1, mk1, gath1, outv1, gsem1, osem1))

        zero16i = jnp.zeros((16,), jnp.int32)

        def zf_body(i, c):
            plsc.store_scatter(idx0, [i * 16 + iota], zero16i)
            return c

        lax.fori_loop(0, M // 16, zf_body, 0, unroll=False)

        for l in range(N_LOC):
            K = 128 >> l
            S1 = _LOC_S1[l]
            S2 = S1 * S1
            lbase = _LOC_BASE[l]
            ng = (S1 + 15) // 16
            bpp, npc = _STAGE[l]
            per_b = 3 * ng * 16

            def stage_body(pc, c, K=K, S1=S1, S2=S2, lbase=lbase, ng=ng,
                           bpp=bpp, per_b=per_b):
                for bi in range(bpp):
                    b = jnp.minimum(pc * bpp + bi, S1 - 1)
                    for p in range(3):
                        for ag in range(ng):
                            a = jnp.minimum(ag * 16 + iota, S1 - 1)
                            r = p * RR + a * K + b * (K * R)
                            r = jnp.minimum(r, 3 * RR - 1)
                            e = (bi * 3 + p) * (ng * 16) + ag * 16 + iota
                            plsc.store_scatter(idx0, [e], r >> 2)
                            plsc.store_scatter(off0, [e], (r & 3) * 2)
                pltpu.async_copy(rows_h.at[idx0], gath0, gsem0).wait()
                for bi in range(bpp):
                    b = jnp.minimum(pc * bpp + bi, S1 - 1)
                    for p in range(3):
                        for ag in range(ng):
                            a = jnp.minimum(ag * 16 + iota, S1 - 1)
                            e = (bi * 3 + p) * (ng * 16) + ag * 16 + iota
                            offv = off0[pl.ds((bi * 3 + p) * (ng * 16)
                                              + ag * 16, 16)]
                            f0 = plsc.load_gather(gath0, [e, offv])
                            f1 = plsc.load_gather(gath0, [e, offv + 1])
                            lidx2 = (lbase + p * S2 + a + b * S1) * 2
                            plsc.store_scatter(loc_v, [lidx2], f0)
                            plsc.store_scatter(loc_v, [lidx2 + 1], f1)
                return c

            lax.fori_loop(0, npc, stage_body, 0, unroll=False)

            for ag in range(ng):
                a = jnp.minimum(ag * 16 + iota, S1 - 1)
                lidx2 = (lbase + 2 * S2 + a + (S1 - 1) * S1) * 2
                plsc.store_scatter(loc_v, [lidx2], last16)
                plsc.store_scatter(loc_v, [lidx2 + 1], last16)

        def make_s1(l, q, c0):
            K = 128 >> l
            sm1 = float(2048 // K - 1)
            idx_v, off_v, mk_v = bufs[q][0], bufs[q][1], bufs[q][2]

            def s1_body(g, c):
                o = c0 + g * 16
                x = pxw[pl.ds(o, 16)]
                y = pyw[pl.ds(o, 16)]
                z = pzw[pl.ds(o, 16)]

                def quant(p):
                    q0 = (p * sm1 + 0.5).astype(jnp.int32) * K
                    return q0, q0 + K

                qx0, qx1 = quant(x)
                qy0, qy1 = quant(y)
                qz0, qz1 = quant(z)
                rz0 = qz0 * R
                rz1 = qz1 * R
                ry0 = qy0 * R
                ry1 = qy1 * R
                rowvals = [
                    qy0 + rz0, qy1 + rz0, qy0 + rz1, qy1 + rz1,
                    RR + qx0 + rz0, RR + qx1 + rz0,
                    RR + qx0 + rz1, RR + qx1 + rz1,
                    2 * RR + qx0 + ry0, 2 * RR + qx1 + ry0,
                    2 * RR + qx0 + ry1, 2 * RR + qx1 + ry1,
                ]
                brow = g * 192
                for i, rv in enumerate(rowvals):
                    addr = brow + i * 16 + iota
                    if i >= 8:
                        mk = jnp.where(rv >= 3 * RR, 1.0, 0.0)
                        plsc.store_scatter(
                            mk_v, [g * 64 + (i - 8) * 16 + iota], mk)
                        rv = jnp.minimum(rv, 3 * RR - 1)
                    plsc.store_scatter(idx_v, [addr], rv >> 2)
                    plsc.store_scatter(off_v, [addr], (rv & 3) * 2)
                return c

            lax.fori_loop(0, G, s1_body, 0, unroll=False)

        def make_s2(l, q, c0, out_v):
            K = 128 >> l
            sm1 = float(2048 // K - 1)
            off_v, mk_v, gath_v = bufs[q][1], bufs[q][2], bufs[q][3]

            def s2_body(g, c):
                o = c0 + g * 16
                x = pxw[pl.ds(o, 16)]
                y = pyw[pl.ds(o, 16)]
                z = pzw[pl.ds(o, 16)]

                def frac(p):
                    ps = p * sm1 + 0.5
                    return ps - ps.astype(jnp.int32).astype(jnp.float32)

                fx = frac(x)
                fy = frac(y)
                fz = frac(z)
                w = [(1.0 - fx, fx), (1.0 - fy, fy), (1.0 - fz, fz)]
                brow = g * 192
                accs = []
                for p, (wa, wb) in enumerate([(w[1], w[2]),
                                              (w[0], w[2]),
                                              (w[0], w[1])]):
                    a0 = jnp.zeros((16,), jnp.float32)
                    a1 = jnp.zeros((16,), jnp.float32)
                    for corner in range(4):
                        ww = wa[corner & 1] * wb[(corner >> 1) & 1]
                        kbase = brow + (p * 4 + corner) * 16
                        rvec = kbase + iota
                        offv = off_v[pl.ds(kbase, 16)]
                        f0 = plsc.load_gather(gath_v, [rvec, offv])
                        f1 = plsc.load_gather(gath_v, [rvec, offv + 1])
                        if p == 2:
                            m = mk_v[pl.ds(g * 64 + corner * 16, 16)]
                            f0 = jnp.where(m > 0.5, last16, f0)
                        a0 = a0 + ww * f0
                        a1 = a1 + ww * f1
                    accs.append((a0, a1))
                s0 = accs[0][0] * accs[1][0] * accs[2][0]
                s1 = accs[0][1] * accs[1][1] * accs[2][1]
                pt64 = (g * 16 + iota) * OUT_D + l * 8
                cols = [accs[0][0], accs[0][1], accs[1][0], accs[1][1],
                        accs[2][0], accs[2][1], s0, s1]
                for j, v in enumerate(cols):
                    plsc.store_scatter(out_v, [pt64 + j], v)
                return c

            lax.fori_loop(0, G, s2_body, 0, unroll=False)

        def make_s2_local(l, c0, out_v):
            K = 128 >> l
            sm1 = float(2048 // K - 1)
            S1 = _LOC_S1[l]
            S2 = S1 * S1
            lbase = _LOC_BASE[l]

            def s2_body(g, c):
                o = c0 + g * 16
                x = pxw[pl.ds(o, 16)]
                y = pyw[pl.ds(o, 16)]
                z = pzw[pl.ds(o, 16)]

                def qf(p):
                    ps = p * sm1 + 0.5
                    gi = ps.astype(jnp.int32)
                    return gi, ps - gi.astype(jnp.float32)

                gx, fx = qf(x)
                gy, fy = qf(y)
                gz, fz = qf(z)
                w = [(1.0 - fx, fx), (1.0 - fy, fy), (1.0 - fz, fz)]
                gs = [gx, gy, gz]
                accs = []
                for p, (ai, bi_) in enumerate([(1, 2), (0, 2), (0, 1)]):
                    wa, wb = w[ai], w[bi_]
                    ca = lbase + p * S2 + gs[ai]
                    cb0 = gs[bi_] * S1
                    a0 = jnp.zeros((16,), jnp.float32)
                    a1 = jnp.zeros((16,), jnp.float32)
                    for corner in range(4):
                        ww = wa[corner & 1] * wb[(corner >> 1) & 1]
                        lidx = ca + (corner & 1) + cb0 + ((corner >> 1) & 1) * S1
                        lidx2 = lidx * 2
                        f0 = plsc.load_gather(loc_v, [lidx2])
                        f1 = plsc.load_gather(loc_v, [lidx2 + 1])
                        a0 = a0 + ww * f0
                        a1 = a1 + ww * f1
                    accs.append((a0, a1))
                s0 = accs[0][0] * accs[1][0] * accs[2][0]
                s1 = accs[0][1] * accs[1][1] * accs[2][1]
                pt64 = (g * 16 + iota) * OUT_D + l * 8
                cols = [accs[0][0], accs[0][1], accs[1][0], accs[1][1],
                        accs[2][0], accs[2][1], s0, s1]
                for j, v in enumerate(cols):
                    plsc.store_scatter(out_v, [pt64 + j], v)
                return c

            lax.fori_loop(0, G, s2_body, 0, unroll=False)

        def start_gather(q):
            return pltpu.async_copy(rows_h.at[bufs[q][0]], bufs[q][3],
                                    bufs[q][5])

        def pair_body(cj, carry):
            for sub in range(2):
                ci = cj * 2 + sub
                c0 = ci * CHUNK
                out_v, osem = bufs[sub][4], bufs[sub][6]
                dst = out_h.at[pl.ds((base + c0) * OUT_D, CHUNK * OUT_D)]

                make_s1(N_LOC, 0, c0)
                cps = [start_gather(0), None]
                for l in range(N_LOC):
                    make_s2_local(l, c0, out_v)
                for l in range(N_LOC, LEVELS):
                    q = (l - N_LOC) & 1
                    if l + 1 < LEVELS:
                        make_s1(l + 1, 1 - q, c0)
                    cps[q].wait()
                    if l + 1 < LEVELS:
                        cps[1 - q] = start_gather(1 - q)
                    make_s2(l, q, c0, out_v)

                pltpu.sync_copy(out_v, dst)
            return carry

        lax.fori_loop(0, n_pairs, pair_body, 0, unroll=False)

    return enc(px, py, pz, rows, lastv)


def kernel(positions, plane_embedding):
    px = positions[:, 0]
    py = positions[:, 1]
    pz = positions[:, 2]
    lastv = jnp.full((16,), plane_embedding[-1], jnp.float32)
    flat = _encode_sc(px, py, pz, plane_embedding.reshape(-1, 8), lastv)
    return flat.reshape(positions.shape[0], OUT_D)


# speedup vs baseline: 5.4800x; 5.4800x over previous
import functools

import jax
import jax.numpy as jnp
from jax import lax
from jax.experimental import pallas as pl
from jax.experimental.pallas import tpu as pltpu
from jax.experimental.pallas import tpu_sc as plsc

R = 2048
RR = R * R
LEVELS = 8
OUT_D = 64

NC = 2
NS = 16
NW = NC * NS

CHUNK = 256
G = CHUNK // 16
M = G * 12 * 16


def _encode_sc(px, py, pz, rows, lastv):
    n = px.shape[0]
    per_w = n // NW
    n_pairs = per_w // (2 * CHUNK)
    mesh = plsc.VectorSubcoreMesh(core_axis_name="c", subcore_axis_name="s")

    buf_types = [
        pltpu.VMEM((M,), jnp.int32),
        pltpu.VMEM((M,), jnp.int32),
        pltpu.VMEM((G * 4 * 16,), jnp.float32),
        pltpu.VMEM((M, 8), jnp.float32),
        pltpu.VMEM((CHUNK * OUT_D,), jnp.float32),
    ]

    @functools.partial(
        pl.kernel,
        mesh=mesh,
        out_type=jax.ShapeDtypeStruct((n * OUT_D,), jnp.float32),
        compiler_params=pltpu.CompilerParams(
            needs_layout_passes=False, use_tc_tiling_on_sc=False),
        scratch_types=[
            pltpu.VMEM((per_w,), jnp.float32),
            pltpu.VMEM((per_w,), jnp.float32),
            pltpu.VMEM((per_w,), jnp.float32),
            pltpu.VMEM((16,), jnp.float32),
        ] + buf_types + buf_types + [
            pltpu.SemaphoreType.DMA,
            pltpu.SemaphoreType.DMA,
            pltpu.SemaphoreType.DMA,
            pltpu.SemaphoreType.DMA,
        ],
    )
    def enc(px_h, py_h, pz_h, rows_h, lastv_h, out_h,
            pxw, pyw, pzw, lastv_v,
            idx0, off0, mk0, gath0, outv0,
            idx1, off1, mk1, gath1, outv1,
            gsem0, gsem1, osem0, osem1):
        wid = lax.axis_index("s") * NC + lax.axis_index("c")
        base = wid * per_w
        iota = lax.broadcasted_iota(jnp.int32, (16,), 0)
        pltpu.sync_copy(lastv_h, lastv_v)
        pltpu.sync_copy(px_h.at[pl.ds(base, per_w)], pxw)
        pltpu.sync_copy(py_h.at[pl.ds(base, per_w)], pyw)
        pltpu.sync_copy(pz_h.at[pl.ds(base, per_w)], pzw)
        last16 = lastv_v[...]

        bufs = ((idx0, off0, mk0, gath0, outv0, gsem0, osem0),
                (idx1, off1, mk1, gath1, outv1, gsem1, osem1))

        def make_s1(l, q, c0):
            K = 128 >> l
            sm1 = float(2048 // K - 1)
            idx_v, off_v, mk_v = bufs[q][0], bufs[q][1], bufs[q][2]

            def s1_body(g, c):
                o = c0 + g * 16
                x = pxw[pl.ds(o, 16)]
                y = pyw[pl.ds(o, 16)]
                z = pzw[pl.ds(o, 16)]

                def quant(p):
                    q0 = (p * sm1 + 0.5).astype(jnp.int32) * K
                    return q0, q0 + K

                qx0, qx1 = quant(x)
                qy0, qy1 = quant(y)
                qz0, qz1 = quant(z)
                rz0 = qz0 * R
                rz1 = qz1 * R
                ry0 = qy0 * R
                ry1 = qy1 * R
                rowvals = [
                    qy0 + rz0, qy1 + rz0, qy0 + rz1, qy1 + rz1,
                    RR + qx0 + rz0, RR + qx1 + rz0,
                    RR + qx0 + rz1, RR + qx1 + rz1,
                    2 * RR + qx0 + ry0, 2 * RR + qx1 + ry0,
                    2 * RR + qx0 + ry1, 2 * RR + qx1 + ry1,
                ]
                brow = g * 192
                for i, rv in enumerate(rowvals):
                    addr = brow + i * 16 + iota
                    if i >= 8:
                        mk = jnp.where(rv >= 3 * RR, 1.0, 0.0)
                        plsc.store_scatter(
                            mk_v, [g * 64 + (i - 8) * 16 + iota], mk)
                        rv = jnp.minimum(rv, 3 * RR - 1)
                    plsc.store_scatter(idx_v, [addr], rv >> 2)
                    plsc.store_scatter(off_v, [addr], (rv & 3) * 2)
                return c

            lax.fori_loop(0, G, s1_body, 0, unroll=False)

        def make_s2(l, q, c0, out_v):
            K = 128 >> l
            sm1 = float(2048 // K - 1)
            off_v, mk_v, gath_v = bufs[q][1], bufs[q][2], bufs[q][3]

            def s2_body(g, c):
                o = c0 + g * 16
                x = pxw[pl.ds(o, 16)]
                y = pyw[pl.ds(o, 16)]
                z = pzw[pl.ds(o, 16)]

                def frac(p):
                    ps = p * sm1 + 0.5
                    return ps - ps.astype(jnp.int32).astype(jnp.float32)

                fx = frac(x)
                fy = frac(y)
                fz = frac(z)
                w = [(1.0 - fx, fx), (1.0 - fy, fy), (1.0 - fz, fz)]
                brow = g * 192
                accs = []
                for p, (wa, wb) in enumerate([(w[1], w[2]),
                                              (w[0], w[2]),
                                              (w[0], w[1])]):
                    a0 = jnp.zeros((16,), jnp.float32)
                    a1 = jnp.zeros((16,), jnp.float32)
                    for corner in range(4):
                        ww = wa[corner & 1] * wb[(corner >> 1) & 1]
                        kbase = brow + (p * 4 + corner) * 16
                        rvec = kbase + iota
                        offv = off_v[pl.ds(kbase, 16)]
                        f0 = plsc.load_gather(gath_v, [rvec, offv])
                        f1 = plsc.load_gather(gath_v, [rvec, offv + 1])
                        if p == 2:
                            m = mk_v[pl.ds(g * 64 + corner * 16, 16)]
                            f0 = jnp.where(m > 0.5, last16, f0)
                        a0 = a0 + ww * f0
                        a1 = a1 + ww * f1
                    accs.append((a0, a1))
                s0 = accs[0][0] * accs[1][0] * accs[2][0]
                s1 = accs[0][1] * accs[1][1] * accs[2][1]
                pt64 = (g * 16 + iota) * OUT_D + l * 8
                cols = [accs[0][0], accs[0][1], accs[1][0], accs[1][1],
                        accs[2][0], accs[2][1], s0, s1]
                for j, v in enumerate(cols):
                    plsc.store_scatter(out_v, [pt64 + j], v)
                return c

            lax.fori_loop(0, G, s2_body, 0, unroll=False)

        def start_gather(q):
            return pltpu.async_copy(rows_h.at[bufs[q][0]], bufs[q][3],
                                    bufs[q][5])

        def pair_body(cj, carry):
            for sub in range(2):
                ci = cj * 2 + sub
                c0 = ci * CHUNK
                out_v = bufs[sub][4]
                dst = out_h.at[pl.ds((base + c0) * OUT_D, CHUNK * OUT_D)]

                make_s1(0, 0, c0)
                cps = [start_gather(0), None]
                for l in range(LEVELS):
                    q = l & 1
                    if l + 1 < LEVELS:
                        make_s1(l + 1, 1 - q, c0)
                    cps[q].wait()
                    if l + 1 < LEVELS:
                        cps[1 - q] = start_gather(1 - q)
                    make_s2(l, q, c0, out_v)

                pltpu.sync_copy(out_v, dst)
            return carry

        lax.fori_loop(0, n_pairs, pair_body, 0, unroll=False)

    return enc(px, py, pz, rows, lastv)


def kernel(positions, plane_embedding):
    px = positions[:, 0]
    py = positions[:, 1]
    pz = positions[:, 2]
    lastv = jnp.full((16,), plane_embedding[-1], jnp.float32)
    flat = _encode_sc(px, py, pz, plane_embedding.reshape(-1, 8), lastv)
    return flat.reshape(positions.shape[0], OUT_D)


# speedup vs baseline: 5.4878x; 1.0014x over previous
import functools

import jax
import jax.numpy as jnp
from jax import lax
from jax.experimental import pallas as pl
from jax.experimental.pallas import tpu as pltpu
from jax.experimental.pallas import tpu_sc as plsc

R = 2048
RR = R * R
LEVELS = 8
OUT_D = 64

NC = 2
NS = 16
NW = NC * NS

CHUNK = 256
G = CHUNK // 16
M = G * 12 * 16


def _encode_sc(px, py, pz, rows, lastv):
    n = px.shape[0]
    per_w = n // NW
    n_pairs = per_w // (2 * CHUNK)
    mesh = plsc.VectorSubcoreMesh(core_axis_name="c", subcore_axis_name="s")

    buf_types = [
        pltpu.VMEM((M,), jnp.int32),
        pltpu.VMEM((M,), jnp.int32),
        pltpu.VMEM((G * 4 * 16,), jnp.float32),
        pltpu.VMEM((M, 8), jnp.float32),
        pltpu.VMEM((CHUNK // 2, 128), jnp.float32),
    ]

    @functools.partial(
        pl.kernel,
        mesh=mesh,
        out_type=jax.ShapeDtypeStruct((n // 2, 128), jnp.float32),
        compiler_params=pltpu.CompilerParams(
            needs_layout_passes=False, use_tc_tiling_on_sc=False),
        scratch_types=[
            pltpu.VMEM((per_w,), jnp.float32),
            pltpu.VMEM((per_w,), jnp.float32),
            pltpu.VMEM((per_w,), jnp.float32),
            pltpu.VMEM((16,), jnp.float32),
        ] + buf_types + buf_types + [
            pltpu.SemaphoreType.DMA,
            pltpu.SemaphoreType.DMA,
            pltpu.SemaphoreType.DMA,
            pltpu.SemaphoreType.DMA,
        ],
    )
    def enc(px_h, py_h, pz_h, rows_h, lastv_h, out_h,
            pxw, pyw, pzw, lastv_v,
            idx0, off0, mk0, gath0, outv0,
            idx1, off1, mk1, gath1, outv1,
            gsem0, gsem1, osem0, osem1):
        wid = lax.axis_index("s") * NC + lax.axis_index("c")
        base = wid * per_w
        iota = lax.broadcasted_iota(jnp.int32, (16,), 0)
        pltpu.sync_copy(lastv_h, lastv_v)
        pltpu.sync_copy(px_h.at[pl.ds(base, per_w)], pxw)
        pltpu.sync_copy(py_h.at[pl.ds(base, per_w)], pyw)
        pltpu.sync_copy(pz_h.at[pl.ds(base, per_w)], pzw)
        last16 = lastv_v[...]

        bufs = ((idx0, off0, mk0, gath0, outv0, gsem0, osem0),
                (idx1, off1, mk1, gath1, outv1, gsem1, osem1))

        def make_s1(l, q, c0):
            K = 128 >> l
            sm1 = float(2048 // K - 1)
            idx_v, off_v, mk_v = bufs[q][0], bufs[q][1], bufs[q][2]

            def s1_body(g, c):
                o = c0 + g * 16
                x = pxw[pl.ds(o, 16)]
                y = pyw[pl.ds(o, 16)]
                z = pzw[pl.ds(o, 16)]

                def quant(p):
                    q0 = (p * sm1 + 0.5).astype(jnp.int32) * K
                    return q0, q0 + K

                qx0, qx1 = quant(x)
                qy0, qy1 = quant(y)
                qz0, qz1 = quant(z)
                rz0 = qz0 * R
                rz1 = qz1 * R
                ry0 = qy0 * R
                ry1 = qy1 * R
                rowvals = [
                    qy0 + rz0, qy1 + rz0, qy0 + rz1, qy1 + rz1,
                    RR + qx0 + rz0, RR + qx1 + rz0,
                    RR + qx0 + rz1, RR + qx1 + rz1,
                    2 * RR + qx0 + ry0, 2 * RR + qx1 + ry0,
                    2 * RR + qx0 + ry1, 2 * RR + qx1 + ry1,
                ]
                brow = g * 192
                for i, rv in enumerate(rowvals):
                    addr = brow + i * 16 + iota
                    if i >= 8:
                        mk = jnp.where(rv >= 3 * RR, 1.0, 0.0)
                        plsc.store_scatter(
                            mk_v, [g * 64 + (i - 8) * 16 + iota], mk)
                        rv = jnp.minimum(rv, 3 * RR - 1)
                    plsc.store_scatter(idx_v, [addr], rv >> 2)
                    plsc.store_scatter(off_v, [addr], (rv & 3) * 2)
                return c

            lax.fori_loop(0, G, s1_body, 0, unroll=False)

        def make_s2(l, q, c0, out_v):
            K = 128 >> l
            sm1 = float(2048 // K - 1)
            off_v, mk_v, gath_v = bufs[q][1], bufs[q][2], bufs[q][3]

            def s2_body(g, c):
                o = c0 + g * 16
                x = pxw[pl.ds(o, 16)]
                y = pyw[pl.ds(o, 16)]
                z = pzw[pl.ds(o, 16)]

                def frac(p):
                    ps = p * sm1 + 0.5
                    return ps - ps.astype(jnp.int32).astype(jnp.float32)

                fx = frac(x)
                fy = frac(y)
                fz = frac(z)
                w = [(1.0 - fx, fx), (1.0 - fy, fy), (1.0 - fz, fz)]
                brow = g * 192
                accs = []
                for p, (wa, wb) in enumerate([(w[1], w[2]),
                                              (w[0], w[2]),
                                              (w[0], w[1])]):
                    a0 = jnp.zeros((16,), jnp.float32)
                    a1 = jnp.zeros((16,), jnp.float32)
                    for corner in range(4):
                        ww = wa[corner & 1] * wb[(corner >> 1) & 1]
                        kbase = brow + (p * 4 + corner) * 16
                        rvec = kbase + iota
                        offv = off_v[pl.ds(kbase, 16)]
                        f0 = plsc.load_gather(gath_v, [rvec, offv])
                        f1 = plsc.load_gather(gath_v, [rvec, offv + 1])
                        if p == 2:
                            m = mk_v[pl.ds(g * 64 + corner * 16, 16)]
                            f0 = jnp.where(m > 0.5, last16, f0)
                        a0 = a0 + ww * f0
                        a1 = a1 + ww * f1
                    accs.append((a0, a1))
                s0 = accs[0][0] * accs[1][0] * accs[2][0]
                s1 = accs[0][1] * accs[1][1] * accs[2][1]
                pt = g * 16 + iota
                prow = pt >> 1
                pcol = (pt & 1) * OUT_D + l * 8
                cols = [accs[0][0], accs[0][1], accs[1][0], accs[1][1],
                        accs[2][0], accs[2][1], s0, s1]
                for j, v in enumerate(cols):
                    plsc.store_scatter(out_v, [prow, pcol + j], v)
                return c

            lax.fori_loop(0, G, s2_body, 0, unroll=False)

        def start_gather(q):
            return pltpu.async_copy(rows_h.at[bufs[q][0]], bufs[q][3],
                                    bufs[q][5])

        def pair_body(cj, carry):
            for sub in range(2):
                ci = cj * 2 + sub
                c0 = ci * CHUNK
                out_v = bufs[sub][4]
                dst = out_h.at[pl.ds((base + c0) // 2, CHUNK // 2)]

                make_s1(0, 0, c0)
                cps = [start_gather(0), None]
                for l in range(LEVELS):
                    q = l & 1
                    if l + 1 < LEVELS:
                        make_s1(l + 1, 1 - q, c0)
                    cps[q].wait()
                    if l + 1 < LEVELS:
                        cps[1 - q] = start_gather(1 - q)
                    make_s2(l, q, c0, out_v)

                pltpu.sync_copy(out_v, dst)
            return carry

        lax.fori_loop(0, n_pairs, pair_body, 0, unroll=False)

    return enc(px, py, pz, rows, lastv)


def kernel(positions, plane_embedding):
    px = positions[:, 0]
    py = positions[:, 1]
    pz = positions[:, 2]
    lastv = jnp.full((16,), plane_embedding[-1], jnp.float32)
    wide = _encode_sc(px, py, pz, plane_embedding.reshape(-1, 8), lastv)
    return wide.reshape(positions.shape[0], OUT_D)
